# baseline (speedup 1.0000x reference)
import functools

import jax
import jax.numpy as jnp
from jax import lax
from jax.experimental import pallas as pl
from jax.experimental.pallas import tpu as pltpu
from jax.experimental.pallas import tpu_sc as plsc

_B = 16384
_C = 250
_NC = 2
_NS = 16
_NW = _NC * _NS
_RW = _B // _NW
_L = 16

_TR = 512
_TG = _B // _TR


def _gather_body(y_hbm, perm_hbm, yp_hbm, y_v, p_v, yp_v):
    wid = lax.axis_index("s") * _NC + lax.axis_index("c")
    base = wid * _RW
    pltpu.sync_copy(y_hbm, y_v)
    pltpu.sync_copy(perm_hbm.at[pl.ds(base, _RW)], p_v)

    def grp(g, carry):
        pv = p_v[pl.ds(g * _L, _L)]
        yp_v[pl.ds(g * _L, _L)] = plsc.load_gather(y_v, [pv])
        return carry

    lax.fori_loop(0, _RW // _L, grp, 0)
    pltpu.sync_copy(yp_v, yp_hbm.at[pl.ds(base, _RW)])


def _dense_body(y_ref, yp_ref, c_ref, out_ref):
    yv = y_ref[:]
    ypv = yp_ref[:]
    cv = c_ref[:]
    iota = lax.broadcasted_iota(jnp.int32, (_TR, _C), 1)
    ccol = cv[:, None]
    hit1 = iota == yv[:, None]
    hit2 = iota == ypv[:, None]
    out_ref[...] = (jnp.where(hit1, ccol, 0.0)
                    + jnp.where(hit2, 1.0 - ccol, 0.0))


@functools.partial(jax.jit)
def kernel(y, perm, coeffs):
    y32 = y.astype(jnp.int32)
    mesh = plsc.VectorSubcoreMesh(core_axis_name="c", subcore_axis_name="s")
    gather = pl.kernel(
        _gather_body,
        mesh=mesh,
        out_type=jax.ShapeDtypeStruct((_B,), jnp.int32),
        compiler_params=pltpu.CompilerParams(needs_layout_passes=False),
        scratch_types=[
            pltpu.VMEM((_B,), jnp.int32),
            pltpu.VMEM((_RW,), jnp.int32),
            pltpu.VMEM((_RW,), jnp.int32),
        ],
    )
    yp = gather(y32, perm.astype(jnp.int32))

    y_mixed = pl.pallas_call(
        _dense_body,
        grid=(_TG,),
        in_specs=[
            pl.BlockSpec((_TR,), lambda i: (i,)),
            pl.BlockSpec((_TR,), lambda i: (i,)),
            pl.BlockSpec((_TR,), lambda i: (i,)),
        ],
        out_specs=pl.BlockSpec((_TR, _C), lambda i: (i, 0)),
        out_shape=jax.ShapeDtypeStruct((_B, _C), jnp.float32),
    )(y32, yp, coeffs)

    return (perm, coeffs, y_mixed)


# speedup vs baseline: 1.3901x; 1.3901x over previous
import functools

import jax
import jax.numpy as jnp
from jax import lax
from jax.experimental import pallas as pl
from jax.experimental.pallas import tpu as pltpu
from jax.experimental.pallas import tpu_sc as plsc

_B = 16384
_C = 250
_NC = 2
_NS = 16
_NW = _NC * _NS
_RW = _B // _NW
_RB = 32
_NCH = _RW // _RB
_L = 16


def _mix_body(y_hbm, perm_hbm, c_hbm, out_hbm,
              y_v, p_v, c_v, buf0, buf1,
              sem_y, sem_p, sem_c, sem0, sem1):
    wid = lax.axis_index("s") * _NC + lax.axis_index("c")
    base = wid * _RW

    cp_y = pltpu.async_copy(y_hbm, y_v, sem_y)
    cp_p = pltpu.async_copy(perm_hbm.at[pl.ds(base, _RW)], p_v, sem_p)
    cp_c = pltpu.async_copy(c_hbm.at[pl.ds(base, _RW)], c_v, sem_c)

    zero16 = jnp.zeros((_L,), jnp.float32)
    iota = lax.iota(jnp.int32, _L)
    tail_mask = iota < jnp.int32(_C % _L)

    def _zero_fill(buf):
        def zrow(r, carry):
            for j in range(_C // _L):
                buf[r, pl.ds(j * _L, _L)] = zero16
            rvec = jnp.zeros((_L,), jnp.int32) + r
            plsc.store_scatter(buf, [rvec, (_C // _L) * _L + iota], zero16,
                               mask=tail_mask)
            return carry
        lax.fori_loop(0, _RB, zrow, 0)

    def _rows(ck, g):
        roff = ck * _RB + g * _L
        yv = y_v[pl.ds(base + roff, _L)]
        pv = p_v[pl.ds(roff, _L)]
        yp = plsc.load_gather(y_v, [pv])
        rvec = g * _L + iota
        return roff, rvec, yv, yp

    def _scatter_chunk(buf, ck):
        def grp(g, carry):
            roff, rvec, yv, yp = _rows(ck, g)
            cv = c_v[pl.ds(roff, _L)]
            plsc.store_scatter(buf, [rvec, yv], cv)
            plsc.addupdate_scatter(buf, [rvec, yp], 1.0 - cv)
            return carry
        lax.fori_loop(0, _RB // _L, grp, 0)

    def _zero_chunk(buf, ck):
        def zgrp(g, carry):
            _, rvec, yv, yp = _rows(ck, g)
            plsc.store_scatter(buf, [rvec, yv], zero16)
            plsc.store_scatter(buf, [rvec, yp], zero16)
            return carry
        lax.fori_loop(0, _RB // _L, zgrp, 0)

    bufs = (buf0, buf1)
    sems = (sem0, sem1)

    def _out_slice(ck):
        return out_hbm.at[pl.ds(base + ck * _RB, _RB)]

    _zero_fill(buf0)
    cp_y.wait()
    cp_p.wait()
    cp_c.wait()
    _scatter_chunk(buf0, 0)
    pltpu.async_copy(buf0, _out_slice(0), sem0)

    _zero_fill(buf1)
    _scatter_chunk(buf1, 1)
    pltpu.async_copy(buf1, _out_slice(1), sem1)

    def pair(p, carry):
        ck0 = 2 + p * 2
        for b in range(2):
            ck = ck0 + b
            pltpu.make_async_copy(bufs[b], _out_slice(ck - 2), sems[b]).wait()
            _zero_chunk(bufs[b], ck - 2)
            _scatter_chunk(bufs[b], ck)
            pltpu.async_copy(bufs[b], _out_slice(ck), sems[b])
        return carry

    lax.fori_loop(0, (_NCH - 2) // 2, pair, 0)

    for ck in (_NCH - 2, _NCH - 1):
        b = ck % 2
        pltpu.make_async_copy(bufs[b], _out_slice(ck), sems[b]).wait()


@functools.partial(jax.jit)
def kernel(y, perm, coeffs):
    mesh = plsc.VectorSubcoreMesh(core_axis_name="c", subcore_axis_name="s")
    mix = pl.kernel(
        _mix_body,
        mesh=mesh,
        out_type=jax.ShapeDtypeStruct((_B, _C), jnp.float32),
        compiler_params=pltpu.CompilerParams(needs_layout_passes=False),
        scratch_types=[
            pltpu.VMEM((_B,), jnp.int32),
            pltpu.VMEM((_RW,), jnp.int32),
            pltpu.VMEM((_RW,), jnp.float32),
            pltpu.VMEM((_RB, _C), jnp.float32),
            pltpu.VMEM((_RB, _C), jnp.float32),
            pltpu.SemaphoreType.DMA,
            pltpu.SemaphoreType.DMA,
            pltpu.SemaphoreType.DMA,
            pltpu.SemaphoreType.DMA,
            pltpu.SemaphoreType.DMA,
        ],
    )
    y_mixed = mix(y.astype(jnp.int32), perm.astype(jnp.int32), coeffs)
    return (perm, coeffs, y_mixed)


# speedup vs baseline: 1.4464x; 1.0405x over previous
import functools

import jax
import jax.numpy as jnp
from jax import lax
from jax.experimental import pallas as pl
from jax.experimental.pallas import tpu as pltpu
from jax.experimental.pallas import tpu_sc as plsc

_B = 16384
_C = 250
_NC = 2
_NS = 16
_NW = _NC * _NS
_RW = _B // _NW
_RB = 128
_NCH = _RW // _RB
_L = 16


def _mix_body(y_hbm, perm_hbm, c_hbm, out_hbm,
              y_v, p_v, c_v, buf0, buf1,
              sem_y, sem_p, sem_c, sem0, sem1):
    wid = lax.axis_index("s") * _NC + lax.axis_index("c")
    base = wid * _RW

    cp_y = pltpu.async_copy(y_hbm, y_v, sem_y)
    cp_p = pltpu.async_copy(perm_hbm.at[pl.ds(base, _RW)], p_v, sem_p)
    cp_c = pltpu.async_copy(c_hbm.at[pl.ds(base, _RW)], c_v, sem_c)

    zero16 = jnp.zeros((_L,), jnp.float32)
    iota = lax.iota(jnp.int32, _L)
    tail_mask = iota < jnp.int32(_C % _L)

    def _zero_fill(buf):
        def zrow(r, carry):
            for j in range(_C // _L):
                buf[r, pl.ds(j * _L, _L)] = zero16
            rvec = jnp.zeros((_L,), jnp.int32) + r
            plsc.store_scatter(buf, [rvec, (_C // _L) * _L + iota], zero16,
                               mask=tail_mask)
            return carry
        lax.fori_loop(0, _RB, zrow, 0)

    def _rows(ck, g):
        roff = ck * _RB + g * _L
        yv = y_v[pl.ds(base + roff, _L)]
        pv = p_v[pl.ds(roff, _L)]
        yp = plsc.load_gather(y_v, [pv])
        rvec = g * _L + iota
        return roff, rvec, yv, yp

    def _scatter_chunk(buf, ck):
        def grp(g, carry):
            roff, rvec, yv, yp = _rows(ck, g)
            cv = c_v[pl.ds(roff, _L)]
            plsc.store_scatter(buf, [rvec, yv], cv)
            plsc.addupdate_scatter(buf, [rvec, yp], 1.0 - cv)
            return carry
        lax.fori_loop(0, _RB // _L, grp, 0)

    def _zero_chunk(buf, ck):
        def zgrp(g, carry):
            _, rvec, yv, yp = _rows(ck, g)
            plsc.store_scatter(buf, [rvec, yv], zero16)
            plsc.store_scatter(buf, [rvec, yp], zero16)
            return carry
        lax.fori_loop(0, _RB // _L, zgrp, 0)

    bufs = (buf0, buf1)
    sems = (sem0, sem1)

    def _out_slice(ck):
        return out_hbm.at[pl.ds(base + ck * _RB, _RB)]

    _zero_fill(buf0)
    cp_y.wait()
    cp_p.wait()
    cp_c.wait()
    _scatter_chunk(buf0, 0)
    pltpu.async_copy(buf0, _out_slice(0), sem0)

    _zero_fill(buf1)
    _scatter_chunk(buf1, 1)
    pltpu.async_copy(buf1, _out_slice(1), sem1)

    def pair(p, carry):
        ck0 = 2 + p * 2
        for b in range(2):
            ck = ck0 + b
            pltpu.make_async_copy(bufs[b], _out_slice(ck - 2), sems[b]).wait()
            _zero_chunk(bufs[b], ck - 2)
            _scatter_chunk(bufs[b], ck)
            pltpu.async_copy(bufs[b], _out_slice(ck), sems[b])
        return carry

    lax.fori_loop(0, (_NCH - 2) // 2, pair, 0)

    for ck in (_NCH - 2, _NCH - 1):
        b = ck % 2
        pltpu.make_async_copy(bufs[b], _out_slice(ck), sems[b]).wait()


@functools.partial(jax.jit)
def kernel(y, perm, coeffs):
    mesh = plsc.VectorSubcoreMesh(core_axis_name="c", subcore_axis_name="s")
    mix = pl.kernel(
        _mix_body,
        mesh=mesh,
        out_type=jax.ShapeDtypeStruct((_B, _C), jnp.float32),
        compiler_params=pltpu.CompilerParams(needs_layout_passes=False),
        scratch_types=[
            pltpu.VMEM((_B,), jnp.int32),
            pltpu.VMEM((_RW,), jnp.int32),
            pltpu.VMEM((_RW,), jnp.float32),
            pltpu.VMEM((_RB, _C), jnp.float32),
            pltpu.VMEM((_RB, _C), jnp.float32),
            pltpu.SemaphoreType.DMA,
            pltpu.SemaphoreType.DMA,
            pltpu.SemaphoreType.DMA,
            pltpu.SemaphoreType.DMA,
            pltpu.SemaphoreType.DMA,
        ],
    )
    y_mixed = mix(y.astype(jnp.int32), perm.astype(jnp.int32), coeffs)
    return (perm, coeffs, y_mixed)
